# baseline (speedup 1.0000x reference)
import jax
import jax.numpy as jnp
from jax.experimental import pallas as pl
from jax.experimental.pallas import tpu as pltpu

N = 10000
K_NEIGH = 32
D_FEAT = 128
IN_DIM = 2 * D_FEAT
OUT_DIM = 128

BLOCK_N = 400
NBUF = 3


def _agg_body(feat_ref, emb_hbm, w_ref, b_ref, out_ref, ebuf, esem):
    i = pl.program_id(0)
    nsteps = pl.num_programs(0)

    def _start(j):
        slot = jax.lax.rem(j, NBUF)
        pltpu.make_async_copy(
            emb_hbm.at[pl.ds(j * BLOCK_N, BLOCK_N)],
            ebuf.at[slot],
            esem.at[slot],
        ).start()

    @pl.when(i == 0)
    def _():
        _start(0)
        _start(1)

    @pl.when(i + 2 < nsteps)
    def _():
        _start(i + 2)

    slot = jax.lax.rem(i, NBUF)
    pltpu.make_async_copy(
        emb_hbm.at[pl.ds(i * BLOCK_N, BLOCK_N)],
        ebuf.at[slot],
        esem.at[slot],
    ).wait()

    emb = ebuf[slot]
    m = jnp.mean(emb, axis=1)
    x = jnp.concatenate([feat_ref[...], m], axis=1)
    y = jnp.dot(x, w_ref[...], preferred_element_type=jnp.float32)
    out_ref[...] = jnp.maximum(y + b_ref[...], 0.0)


def kernel(features, embedding_look_up, kernel, bias_weights):
    bias2d = bias_weights.reshape(1, OUT_DIM)
    return pl.pallas_call(
        _agg_body,
        grid=(N // BLOCK_N,),
        in_specs=[
            pl.BlockSpec((BLOCK_N, D_FEAT), lambda i: (i, 0)),
            pl.BlockSpec(memory_space=pltpu.MemorySpace.HBM),
            pl.BlockSpec((IN_DIM, OUT_DIM), lambda i: (0, 0)),
            pl.BlockSpec((1, OUT_DIM), lambda i: (0, 0)),
        ],
        out_specs=pl.BlockSpec((BLOCK_N, OUT_DIM), lambda i: (i, 0)),
        out_shape=jax.ShapeDtypeStruct((N, OUT_DIM), jnp.float32),
        scratch_shapes=[
            pltpu.VMEM((NBUF, BLOCK_N, K_NEIGH, D_FEAT), jnp.float32),
            pltpu.SemaphoreType.DMA((NBUF,)),
        ],
        compiler_params=pltpu.CompilerParams(
            dimension_semantics=("arbitrary",),
        ),
    )(features, embedding_look_up, kernel, bias2d)


# speedup vs baseline: 1.0289x; 1.0289x over previous
import jax
import jax.numpy as jnp
from jax.experimental import pallas as pl
from jax.experimental.pallas import tpu as pltpu

N = 10000
K_NEIGH = 32
D_FEAT = 128
IN_DIM = 2 * D_FEAT
OUT_DIM = 128

BLOCK_N = 400


def _agg_body(feat_ref, emb_ref, w_ref, b_ref, out_ref):
    emb = emb_ref[...]
    m = jnp.mean(emb, axis=1)
    x = jnp.concatenate([feat_ref[...], m], axis=1)
    y = jnp.dot(x, w_ref[...], preferred_element_type=jnp.float32)
    out_ref[...] = jnp.maximum(y + b_ref[...], 0.0)


def kernel(features, embedding_look_up, kernel, bias_weights):
    bias2d = bias_weights.reshape(1, OUT_DIM)
    return pl.pallas_call(
        _agg_body,
        grid=(N // BLOCK_N,),
        in_specs=[
            pl.BlockSpec((BLOCK_N, D_FEAT), lambda i: (i, 0)),
            pl.BlockSpec((BLOCK_N, K_NEIGH, D_FEAT), lambda i: (i, 0, 0)),
            pl.BlockSpec((IN_DIM, OUT_DIM), lambda i: (0, 0)),
            pl.BlockSpec((1, OUT_DIM), lambda i: (0, 0)),
        ],
        out_specs=pl.BlockSpec((BLOCK_N, OUT_DIM), lambda i: (i, 0)),
        out_shape=jax.ShapeDtypeStruct((N, OUT_DIM), jnp.float32),
        compiler_params=pltpu.CompilerParams(
            dimension_semantics=("parallel",),
        ),
    )(features, embedding_look_up, kernel, bias2d)
